# baseline (speedup 1.0000x reference)
import functools

import jax
import jax.numpy as jnp
from jax import lax
from jax.experimental import pallas as pl
from jax.experimental.pallas import tpu as pltpu
from jax.experimental.pallas import tpu_sc as plsc

N_NODES = 10000
N_EDGES = 160000
H_SIZE = 256
HALF = 128

NPAD = 10240
ROWS_PER_SUB = NPAD // 16
LAST_ROWS = N_NODES - 15 * ROWS_PER_SUB
EDGES_PER_SUB = N_EDGES // 16
EPAD = 240
K = 128
NCHUNK = (EDGES_PER_SUB + EPAD) // K


def _sc_body(h2_hbm, gidx_hbm, sidx_hbm, zeros_hbm, out_hbm,
             acc_sh, gidx_v, sidx_v, rows_v, sem):
    c = lax.axis_index("c")
    s = lax.axis_index("s")
    row0 = s * ROWS_PER_SUB

    pltpu.sync_copy(zeros_hbm, acc_sh.at[pl.ds(row0, ROWS_PER_SUB)])
    plsc.subcore_barrier()

    def body(i, carry):
        pltpu.sync_copy(gidx_hbm.at[c, s, i], gidx_v)
        pltpu.sync_copy(sidx_hbm.at[s, i], sidx_v)
        pltpu.async_copy(h2_hbm.at[gidx_v], rows_v, sem).wait()
        pltpu.sync_copy(rows_v, acc_sh.at[sidx_v], add=True)
        return carry

    lax.fori_loop(0, NCHUNK, body, 0)
    plsc.subcore_barrier()

    out0 = c * N_NODES + row0

    @pl.when(s < 15)
    def _():
        pltpu.sync_copy(acc_sh.at[pl.ds(row0, ROWS_PER_SUB)],
                        out_hbm.at[pl.ds(out0, ROWS_PER_SUB)])

    @pl.when(s == 15)
    def _():
        pltpu.sync_copy(acc_sh.at[pl.ds(row0, LAST_ROWS)],
                        out_hbm.at[pl.ds(out0, LAST_ROWS)])


_sc_segment_sum = functools.partial(
    pl.kernel,
    out_type=jax.ShapeDtypeStruct((2 * N_NODES, HALF), jnp.float32),
    mesh=plsc.VectorSubcoreMesh(core_axis_name="c", subcore_axis_name="s"),
    scratch_types=[
        pltpu.VMEM_SHARED((NPAD, HALF), jnp.float32),
        pltpu.VMEM((K,), jnp.int32),
        pltpu.VMEM((K,), jnp.int32),
        pltpu.VMEM((K, HALF), jnp.float32),
        pltpu.SemaphoreType.DMA,
    ],
)(_sc_body)


ROW_BLK = 1000


def _tc_body(x_ref, ht0_ref, ht1_ref, w_ref, u_ref, b_ref, out_ref):
    ht = jnp.concatenate([ht0_ref[...], ht1_ref[...]], axis=1)
    ruo = (jnp.dot(x_ref[...], w_ref[:, H_SIZE:],
                   preferred_element_type=jnp.float32)
           + jnp.dot(ht, u_ref[:, H_SIZE:],
                     preferred_element_type=jnp.float32)
           + b_ref[:, H_SIZE:])
    u = jax.nn.sigmoid(ruo[:, :H_SIZE])
    o = jnp.tanh(ruo[:, H_SIZE:])
    out_ref[...] = o * u + (1.0 - u) * ht


_tc_dense = pl.pallas_call(
    _tc_body,
    out_shape=jax.ShapeDtypeStruct((N_NODES, H_SIZE), jnp.float32),
    grid=(N_NODES // ROW_BLK,),
    in_specs=[
        pl.BlockSpec((ROW_BLK, H_SIZE), lambda i: (i, 0)),
        pl.BlockSpec((ROW_BLK, HALF), lambda i: (i, 0)),
        pl.BlockSpec((ROW_BLK, HALF), lambda i: (i + 10, 0)),
        pl.BlockSpec((H_SIZE, 3 * H_SIZE), lambda i: (0, 0)),
        pl.BlockSpec((H_SIZE, 3 * H_SIZE), lambda i: (0, 0)),
        pl.BlockSpec((1, 3 * H_SIZE), lambda i: (0, 0)),
    ],
    out_specs=pl.BlockSpec((ROW_BLK, H_SIZE), lambda i: (i, 0)),
)


def kernel(x, h, edge_index, W_ruo, U_ruo, b_ruo):
    src = edge_index[0].astype(jnp.int32)
    dst = edge_index[1].astype(jnp.int32)

    g0 = src * 2
    gidx = jnp.stack([g0, g0 + 1]).reshape(2, 16, EDGES_PER_SUB)
    gidx = jnp.pad(gidx, ((0, 0), (0, 0), (0, EPAD)))
    gidx = gidx.reshape(2, 16, NCHUNK, K)
    sidx = jnp.pad(dst.reshape(16, EDGES_PER_SUB), ((0, 0), (0, EPAD)),
                   constant_values=N_NODES)
    sidx = sidx.reshape(16, NCHUNK, K)

    h2 = h.reshape(2 * N_NODES, HALF)
    zeros = jnp.zeros((ROWS_PER_SUB, HALF), jnp.float32)

    ht_flat = _sc_segment_sum(h2, gidx, sidx, zeros)
    return _tc_dense(x, ht_flat, ht_flat, W_ruo, U_ruo, b_ruo)


# speedup vs baseline: 1.1692x; 1.1692x over previous
import functools

import jax
import jax.numpy as jnp
from jax import lax
from jax.experimental import pallas as pl
from jax.experimental.pallas import tpu as pltpu
from jax.experimental.pallas import tpu_sc as plsc

N_NODES = 10000
N_EDGES = 160000
H_SIZE = 256
HALF = 128

NPAD = 10240
ROWS_PER_SUB = NPAD // 16
LAST_ROWS = N_NODES - 15 * ROWS_PER_SUB
EDGES_PER_SUB = N_EDGES // 16
EPAD = 240
K = 80
NCHUNK = (EDGES_PER_SUB + EPAD) // K


def _sc_body(h2_hbm, gidx_hbm, sidx_hbm, zeros_hbm, out_hbm,
             acc_sh, gring, sidx_v, rows0, rows1,
             sem0, sem1, semi0, semi1):
    c = lax.axis_index("c")
    s = lax.axis_index("s")
    row0 = s * ROWS_PER_SUB

    pltpu.sync_copy(zeros_hbm, acc_sh.at[pl.ds(row0, ROWS_PER_SUB)])
    pltpu.sync_copy(sidx_hbm.at[s], sidx_v)
    plsc.subcore_barrier()

    pltpu.async_copy(gidx_hbm.at[c, s, 0], gring.at[0], semi0)
    pltpu.make_async_copy(gidx_hbm.at[c, s, 0], gring.at[0], semi0).wait()
    pltpu.async_copy(h2_hbm.at[gring.at[0]], rows0, sem0)
    pltpu.async_copy(gidx_hbm.at[c, s, 1], gring.at[1], semi1)

    def body(j, carry):
        i0 = 2 * j
        i1 = i0 + 1
        more = j < NCHUNK // 2 - 1
        pltpu.make_async_copy(h2_hbm.at[gring.at[0]], rows0, sem0).wait()
        pltpu.make_async_copy(gidx_hbm.at[c, s, i1], gring.at[1], semi1).wait()
        pltpu.async_copy(h2_hbm.at[gring.at[1]], rows1, sem1)

        @pl.when(more)
        def _():
            pltpu.async_copy(gidx_hbm.at[c, s, i0 + 2], gring.at[0], semi0)

        pltpu.sync_copy(rows0, acc_sh.at[sidx_v.at[i0]], add=True)
        pltpu.make_async_copy(h2_hbm.at[gring.at[1]], rows1, sem1).wait()

        @pl.when(more)
        def _():
            pltpu.make_async_copy(gidx_hbm.at[c, s, i0 + 2], gring.at[0],
                                  semi0).wait()
            pltpu.async_copy(h2_hbm.at[gring.at[0]], rows0, sem0)
            pltpu.async_copy(gidx_hbm.at[c, s, i1 + 2], gring.at[1], semi1)

        pltpu.sync_copy(rows1, acc_sh.at[sidx_v.at[i1]], add=True)
        return carry

    lax.fori_loop(0, NCHUNK // 2, body, 0)
    plsc.subcore_barrier()

    out0 = c * N_NODES + row0

    @pl.when(s < 15)
    def _():
        pltpu.sync_copy(acc_sh.at[pl.ds(row0, ROWS_PER_SUB)],
                        out_hbm.at[pl.ds(out0, ROWS_PER_SUB)])

    @pl.when(s == 15)
    def _():
        pltpu.sync_copy(acc_sh.at[pl.ds(row0, LAST_ROWS)],
                        out_hbm.at[pl.ds(out0, LAST_ROWS)])


_sc_segment_sum = functools.partial(
    pl.kernel,
    out_type=jax.ShapeDtypeStruct((2 * N_NODES, HALF), jnp.float32),
    mesh=plsc.VectorSubcoreMesh(core_axis_name="c", subcore_axis_name="s"),
    scratch_types=[
        pltpu.VMEM_SHARED((NPAD, HALF), jnp.float32),
        pltpu.VMEM((2, K), jnp.int32),
        pltpu.VMEM((NCHUNK, K), jnp.int32),
        pltpu.VMEM((K, HALF), jnp.float32),
        pltpu.VMEM((K, HALF), jnp.float32),
        pltpu.SemaphoreType.DMA,
        pltpu.SemaphoreType.DMA,
        pltpu.SemaphoreType.DMA,
        pltpu.SemaphoreType.DMA,
    ],
)(_sc_body)


ROW_BLK = 1000


def _tc_body(x_ref, ht0_ref, ht1_ref, w_ref, u_ref, b_ref, out_ref):
    ht = jnp.concatenate([ht0_ref[...], ht1_ref[...]], axis=1)
    ruo = (jnp.dot(x_ref[...], w_ref[:, H_SIZE:],
                   preferred_element_type=jnp.float32)
           + jnp.dot(ht, u_ref[:, H_SIZE:],
                     preferred_element_type=jnp.float32)
           + b_ref[:, H_SIZE:])
    u = jax.nn.sigmoid(ruo[:, :H_SIZE])
    o = jnp.tanh(ruo[:, H_SIZE:])
    out_ref[...] = o * u + (1.0 - u) * ht


_tc_dense = pl.pallas_call(
    _tc_body,
    out_shape=jax.ShapeDtypeStruct((N_NODES, H_SIZE), jnp.float32),
    grid=(N_NODES // ROW_BLK,),
    in_specs=[
        pl.BlockSpec((ROW_BLK, H_SIZE), lambda i: (i, 0)),
        pl.BlockSpec((ROW_BLK, HALF), lambda i: (i, 0)),
        pl.BlockSpec((ROW_BLK, HALF), lambda i: (i + 10, 0)),
        pl.BlockSpec((H_SIZE, 3 * H_SIZE), lambda i: (0, 0)),
        pl.BlockSpec((H_SIZE, 3 * H_SIZE), lambda i: (0, 0)),
        pl.BlockSpec((1, 3 * H_SIZE), lambda i: (0, 0)),
    ],
    out_specs=pl.BlockSpec((ROW_BLK, H_SIZE), lambda i: (i, 0)),
)


def kernel(x, h, edge_index, W_ruo, U_ruo, b_ruo):
    src = edge_index[0].astype(jnp.int32)
    dst = edge_index[1].astype(jnp.int32)

    g0 = src * 2
    gidx = jnp.stack([g0, g0 + 1]).reshape(2, 16, EDGES_PER_SUB)
    gidx = jnp.pad(gidx, ((0, 0), (0, 0), (0, EPAD)))
    gidx = gidx.reshape(2, 16, NCHUNK, K)
    sidx = jnp.pad(dst.reshape(16, EDGES_PER_SUB), ((0, 0), (0, EPAD)),
                   constant_values=N_NODES)
    sidx = sidx.reshape(16, NCHUNK, K)

    h2 = h.reshape(2 * N_NODES, HALF)
    zeros = jnp.zeros((ROWS_PER_SUB, HALF), jnp.float32)

    ht_flat = _sc_segment_sum(h2, gidx, sidx, zeros)
    return _tc_dense(x, ht_flat, ht_flat, W_ruo, U_ruo, b_ruo)


# speedup vs baseline: 1.5430x; 1.3197x over previous
import functools

import jax
import jax.numpy as jnp
from jax import lax
from jax.experimental import pallas as pl
from jax.experimental.pallas import tpu as pltpu
from jax.experimental.pallas import tpu_sc as plsc

N_NODES = 10000
N_EDGES = 160000
H_SIZE = 256
HALF = 128

NPAD = 10240
ROWS_PER_SUB = NPAD // 16
LAST_ROWS = N_NODES - 15 * ROWS_PER_SUB
EDGES_PER_SUB = N_EDGES // 16
K = 80
NCHUNK = EDGES_PER_SUB // K


def _sc_body(h2_hbm, gidx_hbm, sidx_hbm, zeros_hbm, out_hbm,
             acc_sh, gidx_v, sidx_v, rows_v, sem):
    c = lax.axis_index("c")
    s = lax.axis_index("s")
    row0 = s * ROWS_PER_SUB

    pltpu.sync_copy(zeros_hbm, acc_sh.at[pl.ds(row0, ROWS_PER_SUB)])
    plsc.subcore_barrier()

    def body(i, carry):
        pltpu.sync_copy(gidx_hbm.at[c, s, i], gidx_v)
        pltpu.sync_copy(sidx_hbm.at[s, i], sidx_v)
        pltpu.async_copy(h2_hbm.at[gidx_v], rows_v, sem).wait()
        pltpu.sync_copy(rows_v, acc_sh.at[sidx_v], add=True)
        return carry

    lax.fori_loop(0, NCHUNK, body, 0)
    plsc.subcore_barrier()

    out0 = c * N_NODES + row0

    @pl.when(s < 15)
    def _():
        pltpu.sync_copy(acc_sh.at[pl.ds(row0, ROWS_PER_SUB)],
                        out_hbm.at[pl.ds(out0, ROWS_PER_SUB)])

    @pl.when(s == 15)
    def _():
        pltpu.sync_copy(acc_sh.at[pl.ds(row0, LAST_ROWS)],
                        out_hbm.at[pl.ds(out0, LAST_ROWS)])


_sc_segment_sum = functools.partial(
    pl.kernel,
    out_type=jax.ShapeDtypeStruct((2 * N_NODES, HALF), jnp.float32),
    mesh=plsc.VectorSubcoreMesh(core_axis_name="c", subcore_axis_name="s"),
    scratch_types=[
        pltpu.VMEM_SHARED((NPAD, HALF), jnp.float32),
        pltpu.VMEM((K,), jnp.int32),
        pltpu.VMEM((K,), jnp.int32),
        pltpu.VMEM((K, HALF), jnp.float32),
        pltpu.SemaphoreType.DMA,
    ],
)(_sc_body)


ROW_BLK = 1000


def _tc_body(x_ref, ht0_ref, ht1_ref, w_ref, u_ref, b_ref, out_ref):
    ht = jnp.concatenate([ht0_ref[...], ht1_ref[...]], axis=1)
    ruo = (jnp.dot(x_ref[...], w_ref[:, H_SIZE:],
                   preferred_element_type=jnp.float32)
           + jnp.dot(ht, u_ref[:, H_SIZE:],
                     preferred_element_type=jnp.float32)
           + b_ref[:, H_SIZE:])
    u = jax.nn.sigmoid(ruo[:, :H_SIZE])
    o = jnp.tanh(ruo[:, H_SIZE:])
    out_ref[...] = o * u + (1.0 - u) * ht


_tc_dense = pl.pallas_call(
    _tc_body,
    out_shape=jax.ShapeDtypeStruct((N_NODES, H_SIZE), jnp.float32),
    grid=(N_NODES // ROW_BLK,),
    in_specs=[
        pl.BlockSpec((ROW_BLK, H_SIZE), lambda i: (i, 0)),
        pl.BlockSpec((ROW_BLK, HALF), lambda i: (i, 0)),
        pl.BlockSpec((ROW_BLK, HALF), lambda i: (i + 10, 0)),
        pl.BlockSpec((H_SIZE, 3 * H_SIZE), lambda i: (0, 0)),
        pl.BlockSpec((H_SIZE, 3 * H_SIZE), lambda i: (0, 0)),
        pl.BlockSpec((1, 3 * H_SIZE), lambda i: (0, 0)),
    ],
    out_specs=pl.BlockSpec((ROW_BLK, H_SIZE), lambda i: (i, 0)),
)


def kernel(x, h, edge_index, W_ruo, U_ruo, b_ruo):
    src = edge_index[0].astype(jnp.int32)
    dst = edge_index[1].astype(jnp.int32)

    g0 = src * 2
    gidx = jnp.stack([g0, g0 + 1]).reshape(2, 16, NCHUNK, K)
    sidx = dst.reshape(16, NCHUNK, K)

    h2 = h.reshape(2 * N_NODES, HALF)
    zeros = jnp.zeros((ROWS_PER_SUB, HALF), jnp.float32)

    ht_flat = _sc_segment_sum(h2, gidx, sidx, zeros)
    return _tc_dense(x, ht_flat, ht_flat, W_ruo, U_ruo, b_ruo)


# speedup vs baseline: 2.3969x; 1.5534x over previous
import functools

import jax
import jax.numpy as jnp
from jax import lax
from jax.experimental import pallas as pl
from jax.experimental.pallas import tpu as pltpu
from jax.experimental.pallas import tpu_sc as plsc

N_NODES = 10000
N_EDGES = 160000
H_SIZE = 256
HALF = 128

NPAD = 10240
ROWS_PER_SUB = NPAD // 16
LAST_ROWS = N_NODES - 15 * ROWS_PER_SUB
EDGES_PER_SUB = N_EDGES // 16
K = 250
NCHUNK = EDGES_PER_SUB // K


def _sc_body(h2_hbm, gidx_hbm, sidx_hbm, zeros_hbm, out_hbm,
             acc_sh, gidx_v, sidx_v, rows_v, sem):
    c = lax.axis_index("c")
    s = lax.axis_index("s")
    row0 = s * ROWS_PER_SUB

    pltpu.sync_copy(zeros_hbm, acc_sh.at[pl.ds(row0, ROWS_PER_SUB)])
    plsc.subcore_barrier()

    def body(i, carry):
        pltpu.sync_copy(gidx_hbm.at[c, s, i], gidx_v)
        pltpu.sync_copy(sidx_hbm.at[s, i], sidx_v)
        pltpu.async_copy(h2_hbm.at[gidx_v], rows_v, sem).wait()
        pltpu.sync_copy(rows_v, acc_sh.at[sidx_v], add=True)
        return carry

    lax.fori_loop(0, NCHUNK, body, 0)
    plsc.subcore_barrier()

    out0 = c * N_NODES + row0

    @pl.when(s < 15)
    def _():
        pltpu.sync_copy(acc_sh.at[pl.ds(row0, ROWS_PER_SUB)],
                        out_hbm.at[pl.ds(out0, ROWS_PER_SUB)])

    @pl.when(s == 15)
    def _():
        pltpu.sync_copy(acc_sh.at[pl.ds(row0, LAST_ROWS)],
                        out_hbm.at[pl.ds(out0, LAST_ROWS)])


_sc_segment_sum = functools.partial(
    pl.kernel,
    out_type=jax.ShapeDtypeStruct((2 * N_NODES, HALF), jnp.float32),
    mesh=plsc.VectorSubcoreMesh(core_axis_name="c", subcore_axis_name="s"),
    scratch_types=[
        pltpu.VMEM_SHARED((NPAD, HALF), jnp.float32),
        pltpu.VMEM((K,), jnp.int32),
        pltpu.VMEM((K,), jnp.int32),
        pltpu.VMEM((K, HALF), jnp.float32),
        pltpu.SemaphoreType.DMA,
    ],
)(_sc_body)


ROW_BLK = 1000


def _tc_body(x_ref, ht0_ref, ht1_ref, w_ref, u_ref, b_ref, out_ref):
    ht = jnp.concatenate([ht0_ref[...], ht1_ref[...]], axis=1)
    ruo = (jnp.dot(x_ref[...], w_ref[:, H_SIZE:],
                   preferred_element_type=jnp.float32)
           + jnp.dot(ht, u_ref[:, H_SIZE:],
                     preferred_element_type=jnp.float32)
           + b_ref[:, H_SIZE:])
    u = jax.nn.sigmoid(ruo[:, :H_SIZE])
    o = jnp.tanh(ruo[:, H_SIZE:])
    out_ref[...] = o * u + (1.0 - u) * ht


_tc_dense = pl.pallas_call(
    _tc_body,
    out_shape=jax.ShapeDtypeStruct((N_NODES, H_SIZE), jnp.float32),
    grid=(N_NODES // ROW_BLK,),
    in_specs=[
        pl.BlockSpec((ROW_BLK, H_SIZE), lambda i: (i, 0)),
        pl.BlockSpec((ROW_BLK, HALF), lambda i: (i, 0)),
        pl.BlockSpec((ROW_BLK, HALF), lambda i: (i + 10, 0)),
        pl.BlockSpec((H_SIZE, 3 * H_SIZE), lambda i: (0, 0)),
        pl.BlockSpec((H_SIZE, 3 * H_SIZE), lambda i: (0, 0)),
        pl.BlockSpec((1, 3 * H_SIZE), lambda i: (0, 0)),
    ],
    out_specs=pl.BlockSpec((ROW_BLK, H_SIZE), lambda i: (i, 0)),
)


def kernel(x, h, edge_index, W_ruo, U_ruo, b_ruo):
    src = edge_index[0].astype(jnp.int32)
    dst = edge_index[1].astype(jnp.int32)

    g0 = src * 2
    gidx = jnp.stack([g0, g0 + 1]).reshape(2, 16, NCHUNK, K)
    sidx = dst.reshape(16, NCHUNK, K)

    h2 = h.reshape(2 * N_NODES, HALF)
    zeros = jnp.zeros((ROWS_PER_SUB, HALF), jnp.float32)

    ht_flat = _sc_segment_sum(h2, gidx, sidx, zeros)
    return _tc_dense(x, ht_flat, ht_flat, W_ruo, U_ruo, b_ruo)


# speedup vs baseline: 2.7909x; 1.1644x over previous
import functools

import jax
import jax.numpy as jnp
from jax import lax
from jax.experimental import pallas as pl
from jax.experimental.pallas import tpu as pltpu
from jax.experimental.pallas import tpu_sc as plsc

N_NODES = 10000
N_EDGES = 160000
H_SIZE = 256
HALF = 128

NPAD = 10240
ROWS_PER_SUB = NPAD // 16
LAST_ROWS = N_NODES - 15 * ROWS_PER_SUB
EDGES_PER_SUB = N_EDGES // 16
K = 250
NCHUNK = EDGES_PER_SUB // K


def _sc_body(h2_hbm, gidx_hbm, sidx_hbm, zeros_hbm, out_hbm,
             acc_sh, ga, sa, gb, sb, rows_v,
             sem, sga, ssa, sgb, ssb):
    c = lax.axis_index("c")
    s = lax.axis_index("s")
    row0 = s * ROWS_PER_SUB

    pltpu.sync_copy(zeros_hbm, acc_sh.at[pl.ds(row0, ROWS_PER_SUB)])
    pltpu.sync_copy(gidx_hbm.at[c, s, 0], ga)
    pltpu.sync_copy(sidx_hbm.at[s, 0], sa)
    pltpu.async_copy(gidx_hbm.at[c, s, 1], gb, sgb)
    pltpu.async_copy(sidx_hbm.at[s, 1], sb, ssb)
    plsc.subcore_barrier()

    def body(j, carry):
        i0 = 2 * j
        i1 = i0 + 1
        pltpu.async_copy(h2_hbm.at[ga], rows_v, sem).wait()
        pltpu.sync_copy(rows_v, acc_sh.at[sa], add=True)
        nxt0 = jnp.minimum(i0 + 2, NCHUNK - 2)
        pltpu.async_copy(gidx_hbm.at[c, s, nxt0], ga, sga)
        pltpu.async_copy(sidx_hbm.at[s, nxt0], sa, ssa)

        pltpu.make_async_copy(gidx_hbm.at[c, s, i1], gb, sgb).wait()
        pltpu.make_async_copy(sidx_hbm.at[s, i1], sb, ssb).wait()
        pltpu.async_copy(h2_hbm.at[gb], rows_v, sem).wait()
        pltpu.sync_copy(rows_v, acc_sh.at[sb], add=True)
        nxt1 = jnp.minimum(i1 + 2, NCHUNK - 1)
        pltpu.async_copy(gidx_hbm.at[c, s, nxt1], gb, sgb)
        pltpu.async_copy(sidx_hbm.at[s, nxt1], sb, ssb)

        pltpu.make_async_copy(gidx_hbm.at[c, s, nxt0], ga, sga).wait()
        pltpu.make_async_copy(sidx_hbm.at[s, nxt0], sa, ssa).wait()
        return carry

    lax.fori_loop(0, NCHUNK // 2, body, 0)
    pltpu.make_async_copy(gidx_hbm.at[c, s, NCHUNK - 1], gb, sgb).wait()
    pltpu.make_async_copy(sidx_hbm.at[s, NCHUNK - 1], sb, ssb).wait()
    plsc.subcore_barrier()

    out0 = c * N_NODES + row0

    @pl.when(s < 15)
    def _():
        pltpu.sync_copy(acc_sh.at[pl.ds(row0, ROWS_PER_SUB)],
                        out_hbm.at[pl.ds(out0, ROWS_PER_SUB)])

    @pl.when(s == 15)
    def _():
        pltpu.sync_copy(acc_sh.at[pl.ds(row0, LAST_ROWS)],
                        out_hbm.at[pl.ds(out0, LAST_ROWS)])


_sc_segment_sum = functools.partial(
    pl.kernel,
    out_type=jax.ShapeDtypeStruct((2 * N_NODES, HALF), jnp.float32),
    mesh=plsc.VectorSubcoreMesh(core_axis_name="c", subcore_axis_name="s"),
    scratch_types=[
        pltpu.VMEM_SHARED((NPAD, HALF), jnp.float32),
        pltpu.VMEM((K,), jnp.int32),
        pltpu.VMEM((K,), jnp.int32),
        pltpu.VMEM((K,), jnp.int32),
        pltpu.VMEM((K,), jnp.int32),
        pltpu.VMEM((K, HALF), jnp.float32),
        pltpu.SemaphoreType.DMA,
        pltpu.SemaphoreType.DMA,
        pltpu.SemaphoreType.DMA,
        pltpu.SemaphoreType.DMA,
        pltpu.SemaphoreType.DMA,
    ],
)(_sc_body)


ROW_BLK = 1000


def _tc_body(x_ref, ht0_ref, ht1_ref, w_ref, u_ref, b_ref, out_ref):
    ht = jnp.concatenate([ht0_ref[...], ht1_ref[...]], axis=1)
    ruo = (jnp.dot(x_ref[...], w_ref[:, H_SIZE:],
                   preferred_element_type=jnp.float32)
           + jnp.dot(ht, u_ref[:, H_SIZE:],
                     preferred_element_type=jnp.float32)
           + b_ref[:, H_SIZE:])
    u = jax.nn.sigmoid(ruo[:, :H_SIZE])
    o = jnp.tanh(ruo[:, H_SIZE:])
    out_ref[...] = o * u + (1.0 - u) * ht


_tc_dense = pl.pallas_call(
    _tc_body,
    out_shape=jax.ShapeDtypeStruct((N_NODES, H_SIZE), jnp.float32),
    grid=(N_NODES // ROW_BLK,),
    in_specs=[
        pl.BlockSpec((ROW_BLK, H_SIZE), lambda i: (i, 0)),
        pl.BlockSpec((ROW_BLK, HALF), lambda i: (i, 0)),
        pl.BlockSpec((ROW_BLK, HALF), lambda i: (i + 10, 0)),
        pl.BlockSpec((H_SIZE, 3 * H_SIZE), lambda i: (0, 0)),
        pl.BlockSpec((H_SIZE, 3 * H_SIZE), lambda i: (0, 0)),
        pl.BlockSpec((1, 3 * H_SIZE), lambda i: (0, 0)),
    ],
    out_specs=pl.BlockSpec((ROW_BLK, H_SIZE), lambda i: (i, 0)),
)


def kernel(x, h, edge_index, W_ruo, U_ruo, b_ruo):
    src = edge_index[0].astype(jnp.int32)
    dst = edge_index[1].astype(jnp.int32)

    g0 = src * 2
    gidx = jnp.stack([g0, g0 + 1]).reshape(2, 16, NCHUNK, K)
    sidx = dst.reshape(16, NCHUNK, K)

    h2 = h.reshape(2 * N_NODES, HALF)
    zeros = jnp.zeros((ROWS_PER_SUB, HALF), jnp.float32)

    ht_flat = _sc_segment_sum(h2, gidx, sidx, zeros)
    return _tc_dense(x, ht_flat, ht_flat, W_ruo, U_ruo, b_ruo)


# speedup vs baseline: 2.7936x; 1.0010x over previous
import functools

import jax
import jax.numpy as jnp
from jax import lax
from jax.experimental import pallas as pl
from jax.experimental.pallas import tpu as pltpu
from jax.experimental.pallas import tpu_sc as plsc

N_NODES = 10000
N_EDGES = 160000
H_SIZE = 256
HALF = 128

NPAD = 10240
ROWS_PER_SUB = NPAD // 16
LAST_ROWS = N_NODES - 15 * ROWS_PER_SUB
EDGES_PER_SUB = N_EDGES // 16
K = 250
NCHUNK = EDGES_PER_SUB // K


def _sc_body(h2_hbm, gidx_hbm, sidx_hbm, zeros_hbm, out_hbm,
             acc_sh, ga, sa, gb, sb, rows_v,
             sem, sga, ssa, sgb, ssb):
    c = lax.axis_index("c")
    s = lax.axis_index("s")
    row0 = s * ROWS_PER_SUB

    pltpu.sync_copy(zeros_hbm, acc_sh.at[pl.ds(row0, ROWS_PER_SUB)])
    pltpu.sync_copy(gidx_hbm.at[c, s, 0], ga)
    pltpu.sync_copy(sidx_hbm.at[s, 0], sa)
    pltpu.async_copy(gidx_hbm.at[c, s, 1], gb, sgb)
    pltpu.async_copy(sidx_hbm.at[s, 1], sb, ssb)
    plsc.subcore_barrier()

    def body(j, carry):
        i0 = 2 * j
        i1 = i0 + 1
        pltpu.async_copy(h2_hbm.at[ga], rows_v, sem).wait()
        pltpu.sync_copy(rows_v, acc_sh.at[sa], add=True)
        nxt0 = jnp.minimum(i0 + 2, NCHUNK - 2)
        pltpu.async_copy(gidx_hbm.at[c, s, nxt0], ga, sga)
        pltpu.async_copy(sidx_hbm.at[s, nxt0], sa, ssa)

        pltpu.make_async_copy(gidx_hbm.at[c, s, i1], gb, sgb).wait()
        pltpu.make_async_copy(sidx_hbm.at[s, i1], sb, ssb).wait()
        pltpu.async_copy(h2_hbm.at[gb], rows_v, sem).wait()
        pltpu.sync_copy(rows_v, acc_sh.at[sb], add=True)
        nxt1 = jnp.minimum(i1 + 2, NCHUNK - 1)
        pltpu.async_copy(gidx_hbm.at[c, s, nxt1], gb, sgb)
        pltpu.async_copy(sidx_hbm.at[s, nxt1], sb, ssb)

        pltpu.make_async_copy(gidx_hbm.at[c, s, nxt0], ga, sga).wait()
        pltpu.make_async_copy(sidx_hbm.at[s, nxt0], sa, ssa).wait()
        return carry

    lax.fori_loop(0, NCHUNK // 2, body, 0)
    pltpu.make_async_copy(gidx_hbm.at[c, s, NCHUNK - 1], gb, sgb).wait()
    pltpu.make_async_copy(sidx_hbm.at[s, NCHUNK - 1], sb, ssb).wait()
    plsc.subcore_barrier()

    out0 = c * N_NODES + row0

    @pl.when(s < 15)
    def _():
        pltpu.sync_copy(acc_sh.at[pl.ds(row0, ROWS_PER_SUB)],
                        out_hbm.at[pl.ds(out0, ROWS_PER_SUB)])

    @pl.when(s == 15)
    def _():
        pltpu.sync_copy(acc_sh.at[pl.ds(row0, LAST_ROWS)],
                        out_hbm.at[pl.ds(out0, LAST_ROWS)])


_sc_segment_sum = functools.partial(
    pl.kernel,
    out_type=jax.ShapeDtypeStruct((2 * N_NODES, HALF), jnp.float32),
    mesh=plsc.VectorSubcoreMesh(core_axis_name="c", subcore_axis_name="s"),
    scratch_types=[
        pltpu.VMEM_SHARED((NPAD, HALF), jnp.float32),
        pltpu.VMEM((K,), jnp.int32),
        pltpu.VMEM((K,), jnp.int32),
        pltpu.VMEM((K,), jnp.int32),
        pltpu.VMEM((K,), jnp.int32),
        pltpu.VMEM((K, HALF), jnp.float32),
        pltpu.SemaphoreType.DMA,
        pltpu.SemaphoreType.DMA,
        pltpu.SemaphoreType.DMA,
        pltpu.SemaphoreType.DMA,
        pltpu.SemaphoreType.DMA,
    ],
)(_sc_body)


ROW_BLK = 1000


def _tc_body(x_ref, ht0_ref, ht1_ref, w_ref, u_ref, b_ref, out_ref):
    ht = jnp.concatenate([ht0_ref[...], ht1_ref[...]], axis=1)
    ruo = (jnp.dot(x_ref[...].astype(jnp.bfloat16),
                   w_ref[:, H_SIZE:].astype(jnp.bfloat16),
                   preferred_element_type=jnp.float32)
           + jnp.dot(ht.astype(jnp.bfloat16),
                     u_ref[:, H_SIZE:].astype(jnp.bfloat16),
                     preferred_element_type=jnp.float32)
           + b_ref[:, H_SIZE:])
    u = jax.nn.sigmoid(ruo[:, :H_SIZE])
    o = jnp.tanh(ruo[:, H_SIZE:])
    out_ref[...] = o * u + (1.0 - u) * ht


_tc_dense = pl.pallas_call(
    _tc_body,
    out_shape=jax.ShapeDtypeStruct((N_NODES, H_SIZE), jnp.float32),
    grid=(N_NODES // ROW_BLK,),
    in_specs=[
        pl.BlockSpec((ROW_BLK, H_SIZE), lambda i: (i, 0)),
        pl.BlockSpec((ROW_BLK, HALF), lambda i: (i, 0)),
        pl.BlockSpec((ROW_BLK, HALF), lambda i: (i + 10, 0)),
        pl.BlockSpec((H_SIZE, 3 * H_SIZE), lambda i: (0, 0)),
        pl.BlockSpec((H_SIZE, 3 * H_SIZE), lambda i: (0, 0)),
        pl.BlockSpec((1, 3 * H_SIZE), lambda i: (0, 0)),
    ],
    out_specs=pl.BlockSpec((ROW_BLK, H_SIZE), lambda i: (i, 0)),
)


def kernel(x, h, edge_index, W_ruo, U_ruo, b_ruo):
    src = edge_index[0].astype(jnp.int32)
    dst = edge_index[1].astype(jnp.int32)

    g0 = src * 2
    gidx = jnp.stack([g0, g0 + 1]).reshape(2, 16, NCHUNK, K)
    sidx = dst.reshape(16, NCHUNK, K)

    h2 = h.reshape(2 * N_NODES, HALF)
    zeros = jnp.zeros((ROWS_PER_SUB, HALF), jnp.float32)

    ht_flat = _sc_segment_sum(h2, gidx, sidx, zeros)
    return _tc_dense(x, ht_flat, ht_flat, W_ruo, U_ruo, b_ruo)


# speedup vs baseline: 2.8166x; 1.0082x over previous
import functools

import jax
import jax.numpy as jnp
from jax import lax
from jax.experimental import pallas as pl
from jax.experimental.pallas import tpu as pltpu
from jax.experimental.pallas import tpu_sc as plsc

N_NODES = 10000
N_EDGES = 160000
H_SIZE = 256
HALF = 128

NPAD = 10240
ROWS_PER_SUB = NPAD // 16
LAST_ROWS = N_NODES - 15 * ROWS_PER_SUB
EDGES_PER_SUB = N_EDGES // 16
K = 250
NCHUNK = EDGES_PER_SUB // K


def _sc_body(h2_hbm, gidx_hbm, sidx_hbm, zeros_hbm, out_hbm,
             acc_sh, ga, sa, gb, sb, rows_v,
             sem, sga, ssa, sgb, ssb):
    c = lax.axis_index("c")
    s = lax.axis_index("s")
    row0 = s * ROWS_PER_SUB

    pltpu.sync_copy(zeros_hbm, acc_sh.at[pl.ds(row0, ROWS_PER_SUB)])
    pltpu.sync_copy(gidx_hbm.at[c, s, 0], ga)
    pltpu.sync_copy(sidx_hbm.at[s, 0], sa)
    pltpu.async_copy(gidx_hbm.at[c, s, 1], gb, sgb)
    pltpu.async_copy(sidx_hbm.at[s, 1], sb, ssb)
    plsc.subcore_barrier()

    def body(j, carry):
        i0 = 2 * j
        i1 = i0 + 1
        pltpu.async_copy(h2_hbm.at[ga], rows_v, sem).wait()
        pltpu.sync_copy(rows_v, acc_sh.at[sa], add=True)
        nxt0 = jnp.minimum(i0 + 2, NCHUNK - 2)
        pltpu.async_copy(gidx_hbm.at[c, s, nxt0], ga, sga)
        pltpu.async_copy(sidx_hbm.at[s, nxt0], sa, ssa)

        pltpu.make_async_copy(gidx_hbm.at[c, s, i1], gb, sgb).wait()
        pltpu.make_async_copy(sidx_hbm.at[s, i1], sb, ssb).wait()
        pltpu.async_copy(h2_hbm.at[gb], rows_v, sem).wait()
        pltpu.sync_copy(rows_v, acc_sh.at[sb], add=True)
        nxt1 = jnp.minimum(i1 + 2, NCHUNK - 1)
        pltpu.async_copy(gidx_hbm.at[c, s, nxt1], gb, sgb)
        pltpu.async_copy(sidx_hbm.at[s, nxt1], sb, ssb)

        pltpu.make_async_copy(gidx_hbm.at[c, s, nxt0], ga, sga).wait()
        pltpu.make_async_copy(sidx_hbm.at[s, nxt0], sa, ssa).wait()
        return carry

    lax.fori_loop(0, NCHUNK // 2, body, 0)
    pltpu.make_async_copy(gidx_hbm.at[c, s, NCHUNK - 1], gb, sgb).wait()
    pltpu.make_async_copy(sidx_hbm.at[s, NCHUNK - 1], sb, ssb).wait()
    plsc.subcore_barrier()

    out0 = c * N_NODES + row0

    @pl.when(s < 15)
    def _():
        pltpu.sync_copy(acc_sh.at[pl.ds(row0, ROWS_PER_SUB)],
                        out_hbm.at[pl.ds(out0, ROWS_PER_SUB)])

    @pl.when(s == 15)
    def _():
        pltpu.sync_copy(acc_sh.at[pl.ds(row0, LAST_ROWS)],
                        out_hbm.at[pl.ds(out0, LAST_ROWS)])


_sc_segment_sum = functools.partial(
    pl.kernel,
    out_type=jax.ShapeDtypeStruct((2 * N_NODES, HALF), jnp.float32),
    mesh=plsc.VectorSubcoreMesh(core_axis_name="c", subcore_axis_name="s"),
    scratch_types=[
        pltpu.VMEM_SHARED((NPAD, HALF), jnp.float32),
        pltpu.VMEM((K,), jnp.int32),
        pltpu.VMEM((K,), jnp.int32),
        pltpu.VMEM((K,), jnp.int32),
        pltpu.VMEM((K,), jnp.int32),
        pltpu.VMEM((K, HALF), jnp.float32),
        pltpu.SemaphoreType.DMA,
        pltpu.SemaphoreType.DMA,
        pltpu.SemaphoreType.DMA,
        pltpu.SemaphoreType.DMA,
        pltpu.SemaphoreType.DMA,
    ],
)(_sc_body)


ROW_BLK = 2000


def _tc_body(x_ref, ht0_ref, ht1_ref, w_ref, u_ref, b_ref, out_ref):
    ht = jnp.concatenate([ht0_ref[...], ht1_ref[...]], axis=1)
    ruo = (jnp.dot(x_ref[...], w_ref[...],
                   preferred_element_type=jnp.float32)
           + jnp.dot(ht, u_ref[...],
                     preferred_element_type=jnp.float32)
           + b_ref[...])
    u = jax.nn.sigmoid(ruo[:, :H_SIZE])
    o = jnp.tanh(ruo[:, H_SIZE:])
    out_ref[...] = o * u + (1.0 - u) * ht


_tc_dense = pl.pallas_call(
    _tc_body,
    out_shape=jax.ShapeDtypeStruct((N_NODES, H_SIZE), jnp.float32),
    grid=(N_NODES // ROW_BLK,),
    in_specs=[
        pl.BlockSpec((ROW_BLK, H_SIZE), lambda i: (i, 0)),
        pl.BlockSpec((ROW_BLK, HALF), lambda i: (i, 0)),
        pl.BlockSpec((ROW_BLK, HALF), lambda i: (i + N_NODES // ROW_BLK, 0)),
        pl.BlockSpec((H_SIZE, 2 * H_SIZE), lambda i: (0, 0)),
        pl.BlockSpec((H_SIZE, 2 * H_SIZE), lambda i: (0, 0)),
        pl.BlockSpec((1, 2 * H_SIZE), lambda i: (0, 0)),
    ],
    out_specs=pl.BlockSpec((ROW_BLK, H_SIZE), lambda i: (i, 0)),
)


def kernel(x, h, edge_index, W_ruo, U_ruo, b_ruo):
    src = edge_index[0].astype(jnp.int32)
    dst = edge_index[1].astype(jnp.int32)

    g0 = src * 2
    gidx = jnp.stack([g0, g0 + 1]).reshape(2, 16, NCHUNK, K)
    sidx = dst.reshape(16, NCHUNK, K)

    h2 = h.reshape(2 * N_NODES, HALF)
    zeros = jnp.zeros((ROWS_PER_SUB, HALF), jnp.float32)

    ht_flat = _sc_segment_sum(h2, gidx, sidx, zeros)
    return _tc_dense(x, ht_flat, ht_flat, W_ruo[:, H_SIZE:],
                     U_ruo[:, H_SIZE:], b_ruo[:, H_SIZE:])
